# baseline (device time: 17306 ns/iter reference)
import jax
import jax.numpy as jnp
from jax import lax
from jax.experimental import pallas as pl
from jax.experimental.pallas import tpu as pltpu

N_DEV = 4
CHUNK = 1024


def kernel(x, W, labels):
    T, D = x.shape
    Vs = W.shape[1]
    NC = Vs // CHUNK

    def body(x_hbm, w_hbm, lab_ref, out_ref,
             xv, wv, comm_ref, dma_sems, send_sems, recv_sems):
        my = lax.axis_index("i")

        barrier_sem = pltpu.get_barrier_semaphore()
        for j in range(1, N_DEV):
            pl.semaphore_signal(
                barrier_sem, inc=1,
                device_id=((my + j) % N_DEV,),
                device_id_type=pl.DeviceIdType.MESH,
            )

        cp_x = pltpu.make_async_copy(x_hbm, xv, dma_sems.at[2])
        cp_x.start()
        cp_w = [None] * NC
        cp_w[0] = pltpu.make_async_copy(
            w_hbm.at[:, pl.ds(0, CHUNK)], wv.at[0], dma_sems.at[0]
        )
        cp_w[0].start()
        cp_x.wait()
        xb = xv[...].astype(jnp.bfloat16)
        local_idx = lab_ref[...] - my * Vs

        s_acc = jnp.zeros((T, 1), jnp.float32)
        c_acc = jnp.zeros((T, 1), jnp.float32)
        for c in range(NC):
            slot = c % 2
            if c + 1 < NC:
                cp_w[c + 1] = pltpu.make_async_copy(
                    w_hbm.at[:, pl.ds((c + 1) * CHUNK, CHUNK)],
                    wv.at[1 - slot],
                    dma_sems.at[1 - slot],
                )
                cp_w[c + 1].start()
            cp_w[c].wait()
            wb = wv[slot].astype(jnp.bfloat16)
            lg = jnp.dot(xb, wb, preferred_element_type=jnp.float32)
            s_acc = s_acc + jnp.sum(jnp.exp(lg), axis=1, keepdims=True)
            col = lax.broadcasted_iota(jnp.int32, (T, CHUNK), 1) + c * CHUNK
            c_acc = c_acc + jnp.sum(
                jnp.where(col == local_idx, lg, 0.0), axis=1, keepdims=True
            )

        comm_ref[0, :, 0:1] = s_acc
        comm_ref[0, :, 1:2] = c_acc

        pl.semaphore_wait(barrier_sem, N_DEV - 1)

        rdmas = []
        for j in range(1, N_DEV):
            rdma = pltpu.make_async_remote_copy(
                src_ref=comm_ref.at[0],
                dst_ref=comm_ref.at[j],
                send_sem=send_sems.at[j - 1],
                recv_sem=recv_sems.at[j - 1],
                device_id=((my + j) % N_DEV,),
                device_id_type=pl.DeviceIdType.MESH,
            )
            rdma.start()
            rdmas.append(rdma)
        for rdma in rdmas:
            rdma.wait()

        s_tot = s_acc + sum(comm_ref[j, :, 0:1] for j in range(1, N_DEV))
        c_tot = c_acc + sum(comm_ref[j, :, 1:2] for j in range(1, N_DEV))
        out_ref[...] = jnp.log(s_tot) - c_tot

    out = pl.pallas_call(
        body,
        out_shape=jax.ShapeDtypeStruct((T, 1), jnp.float32),
        in_specs=[
            pl.BlockSpec(memory_space=pltpu.MemorySpace.HBM),
            pl.BlockSpec(memory_space=pltpu.MemorySpace.HBM),
            pl.BlockSpec(memory_space=pltpu.VMEM),
        ],
        out_specs=pl.BlockSpec(memory_space=pltpu.VMEM),
        scratch_shapes=[
            pltpu.VMEM((T, D), jnp.float32),
            pltpu.VMEM((2, D, CHUNK), jnp.float32),
            pltpu.VMEM((N_DEV, T, 2), jnp.float32),
            pltpu.SemaphoreType.DMA((3,)),
            pltpu.SemaphoreType.DMA((N_DEV - 1,)),
            pltpu.SemaphoreType.DMA((N_DEV - 1,)),
        ],
        compiler_params=pltpu.CompilerParams(collective_id=0),
    )(x, W, labels.reshape(T, 1))
    return out[:, 0]


# device time: 12036 ns/iter; 1.4379x vs baseline; 1.4379x over previous
import jax
import jax.numpy as jnp
from jax import lax
from jax.experimental import pallas as pl
from jax.experimental.pallas import tpu as pltpu

N_DEV = 4


def kernel(x, W, labels):
    T, D = x.shape
    Vs = W.shape[1]

    def body(x_ref, w_ref, lab_ref, out_ref, comm_ref, send_sems, recv_sems):
        my = lax.axis_index("i")

        barrier_sem = pltpu.get_barrier_semaphore()
        for j in range(1, N_DEV):
            pl.semaphore_signal(
                barrier_sem, inc=1,
                device_id=((my + j) % N_DEV,),
                device_id_type=pl.DeviceIdType.MESH,
            )

        xb = x_ref[...].astype(jnp.bfloat16)
        wb = w_ref[...].astype(jnp.bfloat16)
        lgT = lax.dot_general(
            wb, xb, (((0,), (1,)), ((), ())),
            preferred_element_type=jnp.float32,
        )

        s_row = jnp.sum(jnp.exp(lgT), axis=0, keepdims=True)
        row = lax.broadcasted_iota(jnp.int32, (Vs, T), 0)
        local_idx = lab_ref[...] - my * Vs
        c_row = jnp.sum(
            jnp.where(row == local_idx, lgT, 0.0), axis=0, keepdims=True
        )

        comm_ref[0, 0:1, :] = s_row
        comm_ref[0, 1:2, :] = c_row

        pl.semaphore_wait(barrier_sem, N_DEV - 1)

        rdmas = []
        for j in range(1, N_DEV):
            rdma = pltpu.make_async_remote_copy(
                src_ref=comm_ref.at[0],
                dst_ref=comm_ref.at[j],
                send_sem=send_sems.at[j - 1],
                recv_sem=recv_sems.at[j - 1],
                device_id=((my + j) % N_DEV,),
                device_id_type=pl.DeviceIdType.MESH,
            )
            rdma.start()
            rdmas.append(rdma)
        for rdma in rdmas:
            rdma.wait()

        s_tot = s_row + sum(comm_ref[j, 0:1, :] for j in range(1, N_DEV))
        c_tot = c_row + sum(comm_ref[j, 1:2, :] for j in range(1, N_DEV))
        out_ref[...] = jnp.log(s_tot) - c_tot

    out = pl.pallas_call(
        body,
        out_shape=jax.ShapeDtypeStruct((1, T), jnp.float32),
        in_specs=[pl.BlockSpec(memory_space=pltpu.VMEM)] * 3,
        out_specs=pl.BlockSpec(memory_space=pltpu.VMEM),
        scratch_shapes=[
            pltpu.VMEM((N_DEV, 2, T), jnp.float32),
            pltpu.SemaphoreType.DMA((N_DEV - 1,)),
            pltpu.SemaphoreType.DMA((N_DEV - 1,)),
        ],
        compiler_params=pltpu.CompilerParams(collective_id=0),
    )(x, W, labels.reshape(1, T))
    return out[0]


# device time: 11953 ns/iter; 1.4478x vs baseline; 1.0069x over previous
import jax
import jax.numpy as jnp
from jax import lax
from jax.experimental import pallas as pl
from jax.experimental.pallas import tpu as pltpu

N_DEV = 4


def kernel(x, W, labels):
    T, D = x.shape
    Vs = W.shape[1]

    def body(x_ref, w_ref, lab_ref, out_ref, comm_ref, send_sems, recv_sems):
        my = lax.axis_index("i")

        barrier_sem = pltpu.get_barrier_semaphore()
        for j in range(1, N_DEV):
            pl.semaphore_signal(
                barrier_sem, inc=1,
                device_id=((my + j) % N_DEV,),
                device_id_type=pl.DeviceIdType.MESH,
            )

        xb = x_ref[...].astype(jnp.bfloat16)
        wb = w_ref[...].astype(jnp.bfloat16)
        lgT = lax.dot_general(
            wb, xb, (((0,), (1,)), ((), ())),
            preferred_element_type=jnp.float32,
        )

        s_row = jnp.sum(jnp.exp(lgT), axis=0, keepdims=True)
        row = lax.broadcasted_iota(jnp.int32, (Vs, T), 0)
        local_idx = lab_ref[...] - my * Vs
        c_row = jnp.sum(
            jnp.where(row == local_idx, lgT, 0.0), axis=0, keepdims=True
        )

        comm_ref[0, 0:1, :] = s_row
        comm_ref[0, 1:2, :] = c_row

        pl.semaphore_wait(barrier_sem, N_DEV - 1)

        rdmas = []
        for j in range(1, N_DEV):
            rdma = pltpu.make_async_remote_copy(
                src_ref=comm_ref.at[0],
                dst_ref=comm_ref.at[j],
                send_sem=send_sems.at[j - 1],
                recv_sem=recv_sems.at[j - 1],
                device_id=((my + j) % N_DEV,),
                device_id_type=pl.DeviceIdType.MESH,
            )
            rdma.start()
            rdmas.append(rdma)
        for rdma in rdmas:
            rdma.wait()

        s_tot = s_row + sum(comm_ref[j, 0:1, :] for j in range(1, N_DEV))
        c_tot = c_row + sum(comm_ref[j, 1:2, :] for j in range(1, N_DEV))
        out_ref[...] = (jnp.log(s_tot) - c_tot)[0]

    out = pl.pallas_call(
        body,
        out_shape=jax.ShapeDtypeStruct((T,), jnp.float32),
        in_specs=[pl.BlockSpec(memory_space=pltpu.VMEM)] * 3,
        out_specs=pl.BlockSpec(memory_space=pltpu.VMEM),
        scratch_shapes=[
            pltpu.VMEM((N_DEV, 2, T), jnp.float32),
            pltpu.SemaphoreType.DMA((N_DEV - 1,)),
            pltpu.SemaphoreType.DMA((N_DEV - 1,)),
        ],
        compiler_params=pltpu.CompilerParams(collective_id=0),
    )(x, W, labels.reshape(1, T))
    return out
